# Initial kernel scaffold; baseline (speedup 1.0000x reference)
import jax
import jax.numpy as jnp
from jax.experimental import pallas as pl


def kernel(encoding, pos, batch, graph_sizes, Wup, bup, Wq, bq, Wk, bk, Wv, bv, We, be, Wskip, bskip, Wfc, bfc, Wout, bout):
    raise NotImplementedError("write your pallas kernel here")


# speedup vs baseline: 1.8532x; 1.8532x over previous
import jax
import jax.numpy as jnp
import numpy as np
from jax.experimental import pallas as pl

G, NPG, D, C = 16, 128, 128, 3
NG, NR, H, HD, L = 27, 50, 4, 32, 2
CUTOFF = 2.0
NT = 10

_HIGHEST = jax.lax.Precision.HIGHEST


def _dot(a, b, dims):
    return jax.lax.dot_general(
        a, b, (dims, ((), ())), precision=_HIGHEST,
        preferred_element_type=jnp.float32)


def _decoder_body(enc_ref, pos_ref, gp_ref, offs_ref, coef_ref, wup_ref,
                  bup_ref, wq_ref, bq_ref, wk_ref, bk_ref, wv_ref, bv_ref,
                  we_ref, be_ref, wskip_ref, bskip_ref, wfc_ref, bfc_ref,
                  wout_ref, bout_ref, out_ref):
    f32 = jnp.float32
    gelu = jax.nn.gelu

    pos = pos_ref[...]
    gp = gp_ref[...]

    gridf = gelu(_dot(enc_ref[...].reshape(1, D), wup_ref[...], ((1,), (0,)))
                 + bup_ref[...]).reshape(NG, D)

    gp_t = gp.T
    d2 = ((pos[:, 0:1] - gp_t[0:1, :]) ** 2
          + (pos[:, 1:2] - gp_t[1:2, :]) ** 2
          + (pos[:, 2:3] - gp_t[2:3, :]) ** 2)
    cols = jax.lax.broadcasted_iota(jnp.int32, (NPG, NG), 1).astype(f32)
    d2m = d2
    sels, ws = [], []
    for _ in range(3):
        m = jnp.min(d2m, axis=1, keepdims=True)
        cand = jnp.where(d2m == m, cols, float(NG))
        fidx = jnp.min(cand, axis=1, keepdims=True)
        sel = cols == fidx
        sels.append(sel)
        ws.append(1.0 / (m + 1e-16))
        d2m = jnp.where(sel, 1e30, d2m)
    wtot = ws[0] + ws[1] + ws[2]
    woh = (sels[0].astype(f32) * (ws[0] / wtot)
           + sels[1].astype(f32) * (ws[1] / wtot)
           + sels[2].astype(f32) * (ws[2] / wtot))
    x = gelu(_dot(woh, gridf, ((1,), (0,))))

    offs3 = offs_ref[...].reshape(NR, 1, 1)
    coef3 = coef_ref[...].reshape(1, 1, 1)
    ii = jax.lax.broadcasted_iota(jnp.int32, (NPG, NPG), 0)
    jj = jax.lax.broadcasted_iota(jnp.int32, (NPG, NPG), 1)
    offdiag = ii != jj
    inv_sqrt_hd = 1.0 / np.sqrt(1.0 * HD)

    for l in range(L):
        pos_t = pos.T
        dT2 = ((pos[:, 0:1] - pos_t[0:1, :]) ** 2
               + (pos[:, 1:2] - pos_t[1:2, :]) ** 2
               + (pos[:, 2:3] - pos_t[2:3, :]) ** 2)
        distT = jnp.sqrt(dT2 + 1e-12)
        maskT = (distT < CUTOFF) & offdiag
        maskTf = maskT.astype(f32)
        rbf = jnp.exp(coef3 * (distT[None, :, :] - offs3) ** 2)

        xg = x
        q = _dot(xg, wq_ref[l], ((1,), (0,))) + bq_ref[l]
        k_ = _dot(xg, wk_ref[l], ((1,), (0,))) + bk_ref[l]
        v_ = _dot(xg, wv_ref[l], ((1,), (0,))) + bv_ref[l]
        skip = _dot(xg, wskip_ref[l], ((1,), (0,))) + bskip_ref[l]
        we_l = we_ref[l]
        be_l = be_ref[l]

        heads = []
        for h in range(H):
            sl = slice(h * HD, (h + 1) * HD)
            qh, kh, vh = q[:, sl], k_[:, sl], v_[:, sl]
            weh = we_l[:, sl]
            beh = be_l[:, sl]

            l1T = _dot(kh, qh, ((1,), (1,)))
            aqT = _dot(weh, qh, ((1,), (1,)))
            l2T = jnp.sum(rbf * aqT[:, None, :], axis=0)
            qbeT = _dot(beh, qh, ((1,), (1,)))
            lg = (l1T + l2T + qbeT) * inv_sqrt_hd
            lg = jnp.where(maskT, lg, -1e9)
            lmax = jnp.max(lg, axis=0, keepdims=True)
            ex = jnp.exp(lg - lmax) * maskTf
            den = jnp.sum(ex, axis=0, keepdims=True) + 1e-16
            alT = ex / den

            o1 = _dot(alT, vh, ((0,), (0,)))
            sh = jnp.sum(rbf * alT[None, :, :], axis=1)
            o2 = _dot(sh, weh, ((0,), (0,)))
            asum = _dot(alT, jnp.ones((NPG, 1), f32), ((0,), (0,)))
            heads.append(o1 + o2 + asum * beh)
        attn = jnp.concatenate(heads, axis=1)

        out = attn + skip
        hh = gelu(_dot(out, wfc_ref[l], ((1,), (0,))) + bfc_ref[l])
        xg = xg + hh
        pos = pos + xg[:, :C]
        x = jnp.concatenate([xg[:, :D - C], pos], axis=1)

    proj = _dot(x, wout_ref[...], ((1,), (0,))) + bout_ref[...]
    out_ref[...] = jnp.concatenate([pos, proj], axis=1)


@jax.jit
def kernel(encoding, pos, batch, graph_sizes, Wup, bup, Wq, bq, Wk, bk,
           Wv, bv, We, be, Wskip, bskip, Wfc, bfc, Wout, bout):
    del batch, graph_sizes
    f32 = jnp.float32

    Wup_r = Wup.reshape(D, D, NG).transpose(0, 2, 1).reshape(D, NG * D)
    bup_r = bup.reshape(D, NG).T.reshape(1, NG * D)
    bq_r = bq.reshape(L, 1, D)
    bk_r = bk.reshape(L, 1, D)
    bv_r = bv.reshape(L, 1, D)
    be_r = be.reshape(L, 1, D)
    bskip_r = bskip.reshape(L, 1, D)
    bfc_r = bfc.reshape(L, 1, D)
    bout_r = bout.reshape(1, NT)

    pts = [(xx, yy, zz) for xx in (-1, 0, 1) for yy in (-1, 0, 1)
           for zz in (-1, 0, 1)]
    gp = jnp.asarray(np.array(pts, dtype=np.float32))
    offsets = jnp.linspace(0.0, CUTOFF, NR).astype(f32)
    coeff = (-0.5 / (offsets[1] - offsets[0]) ** 2).reshape(1, 1)
    offs = offsets.reshape(NR, 1)

    const = lambda shape: pl.BlockSpec(shape, lambda g: (0,) * len(shape))
    in_specs = [
        pl.BlockSpec((1, 1, D), lambda g: (g, 0, 0)),
        pl.BlockSpec((NPG, C), lambda g: (g, 0)),
        const((NG, C)),
        const((NR, 1)),
        const((1, 1)),
        const((D, NG * D)),
        const((1, NG * D)),
        const((L, D, D)), const((L, 1, D)),
        const((L, D, D)), const((L, 1, D)),
        const((L, D, D)), const((L, 1, D)),
        const((L, NR, D)), const((L, 1, D)),
        const((L, D, D)), const((L, 1, D)),
        const((L, D, D)), const((L, 1, D)),
        const((D, NT)), const((1, NT)),
    ]
    out = pl.pallas_call(
        _decoder_body,
        grid=(G,),
        in_specs=in_specs,
        out_specs=pl.BlockSpec((NPG, C + NT), lambda g: (g, 0)),
        out_shape=jax.ShapeDtypeStruct((G * NPG, C + NT), f32),
    )(encoding.reshape(G, 1, D), pos, gp, offs, coeff, Wup_r, bup_r, Wq, bq_r, Wk, bk_r,
      Wv, bv_r, We, be_r, Wskip, bskip_r, Wfc, bfc_r, Wout, bout_r)
    return out


# speedup vs baseline: 2.5077x; 1.3532x over previous
import jax
import jax.numpy as jnp
import numpy as np
from jax.experimental import pallas as pl

G, NPG, D, C = 16, 128, 128, 3
NG, NR, H, HD, L = 27, 50, 4, 32, 2
CUTOFF = 2.0
NT = 10

_PREC = jax.lax.Precision.DEFAULT


def _dot(a, b, dims):
    return jax.lax.dot_general(
        a, b, (dims, ((), ())), precision=_PREC,
        preferred_element_type=jnp.float32)


def _decoder_body(enc_ref, pos_ref, gp_ref, offs_ref, coef_ref, wup_ref,
                  bup_ref, wq_ref, bq_ref, wk_ref, bk_ref, wv_ref, bv_ref,
                  we_ref, be_ref, wskip_ref, bskip_ref, wfc_ref, bfc_ref,
                  wout_ref, bout_ref, out_ref):
    f32 = jnp.float32
    gelu = jax.nn.gelu

    pos = pos_ref[...]
    gp = gp_ref[...]

    gridf = gelu(_dot(enc_ref[...].reshape(1, D), wup_ref[...], ((1,), (0,)))
                 + bup_ref[...]).reshape(NG, D)

    gp_t = gp.T
    d2 = ((pos[:, 0:1] - gp_t[0:1, :]) ** 2
          + (pos[:, 1:2] - gp_t[1:2, :]) ** 2
          + (pos[:, 2:3] - gp_t[2:3, :]) ** 2)
    cols = jax.lax.broadcasted_iota(jnp.int32, (NPG, NG), 1).astype(f32)
    d2m = d2
    sels, ws = [], []
    for _ in range(3):
        m = jnp.min(d2m, axis=1, keepdims=True)
        cand = jnp.where(d2m == m, cols, float(NG))
        fidx = jnp.min(cand, axis=1, keepdims=True)
        sel = cols == fidx
        sels.append(sel)
        ws.append(1.0 / (m + 1e-16))
        d2m = jnp.where(sel, 1e30, d2m)
    wtot = ws[0] + ws[1] + ws[2]
    woh = (sels[0].astype(f32) * (ws[0] / wtot)
           + sels[1].astype(f32) * (ws[1] / wtot)
           + sels[2].astype(f32) * (ws[2] / wtot))
    x = gelu(_dot(woh, gridf, ((1,), (0,))))

    offs3 = offs_ref[...].reshape(NR, 1, 1)
    coef3 = coef_ref[...].reshape(1, 1, 1)
    ii = jax.lax.broadcasted_iota(jnp.int32, (NPG, NPG), 0)
    jj = jax.lax.broadcasted_iota(jnp.int32, (NPG, NPG), 1)
    offdiag = ii != jj
    inv_sqrt_hd = 1.0 / np.sqrt(1.0 * HD)

    for l in range(L):
        pos_t = pos.T
        dT2 = ((pos[:, 0:1] - pos_t[0:1, :]) ** 2
               + (pos[:, 1:2] - pos_t[1:2, :]) ** 2
               + (pos[:, 2:3] - pos_t[2:3, :]) ** 2)
        distT = jnp.sqrt(dT2 + 1e-12)
        maskT = (distT < CUTOFF) & offdiag
        maskTf = maskT.astype(f32)
        rbf = jnp.exp(coef3 * (distT[None, :, :] - offs3) ** 2)

        xg = x
        q = _dot(xg, wq_ref[l], ((1,), (0,))) + bq_ref[l]
        k_ = _dot(xg, wk_ref[l], ((1,), (0,))) + bk_ref[l]
        v_ = _dot(xg, wv_ref[l], ((1,), (0,))) + bv_ref[l]
        skip = _dot(xg, wskip_ref[l], ((1,), (0,))) + bskip_ref[l]
        we_l = we_ref[l]
        be_l = be_ref[l]

        heads = []
        for h in range(H):
            sl = slice(h * HD, (h + 1) * HD)
            qh, kh, vh = q[:, sl], k_[:, sl], v_[:, sl]
            weh = we_l[:, sl]
            beh = be_l[:, sl]

            l1T = _dot(kh, qh, ((1,), (1,)))
            aqT = _dot(weh, qh, ((1,), (1,)))
            l2T = jnp.sum(rbf * aqT[:, None, :], axis=0)
            qbeT = _dot(beh, qh, ((1,), (1,)))
            lg = (l1T + l2T + qbeT) * inv_sqrt_hd
            lg = jnp.where(maskT, lg, -1e9)
            lmax = jnp.max(lg, axis=0, keepdims=True)
            ex = jnp.exp(lg - lmax) * maskTf
            den = jnp.sum(ex, axis=0, keepdims=True) + 1e-16
            alT = ex / den

            o1 = _dot(alT, vh, ((0,), (0,)))
            sh = jnp.sum(rbf * alT[None, :, :], axis=1)
            o2 = _dot(sh, weh, ((0,), (0,)))
            asum = _dot(alT, jnp.ones((NPG, 1), f32), ((0,), (0,)))
            heads.append(o1 + o2 + asum * beh)
        attn = jnp.concatenate(heads, axis=1)

        out = attn + skip
        hh = gelu(_dot(out, wfc_ref[l], ((1,), (0,))) + bfc_ref[l])
        xg = xg + hh
        pos = pos + xg[:, :C]
        x = jnp.concatenate([xg[:, :D - C], pos], axis=1)

    proj = _dot(x, wout_ref[...], ((1,), (0,))) + bout_ref[...]
    out_ref[...] = jnp.concatenate([pos, proj], axis=1)


@jax.jit
def kernel(encoding, pos, batch, graph_sizes, Wup, bup, Wq, bq, Wk, bk,
           Wv, bv, We, be, Wskip, bskip, Wfc, bfc, Wout, bout):
    del batch, graph_sizes
    f32 = jnp.float32

    Wup_r = Wup.reshape(D, D, NG).transpose(0, 2, 1).reshape(D, NG * D)
    bup_r = bup.reshape(D, NG).T.reshape(1, NG * D)
    bq_r = bq.reshape(L, 1, D)
    bk_r = bk.reshape(L, 1, D)
    bv_r = bv.reshape(L, 1, D)
    be_r = be.reshape(L, 1, D)
    bskip_r = bskip.reshape(L, 1, D)
    bfc_r = bfc.reshape(L, 1, D)
    bout_r = bout.reshape(1, NT)

    pts = [(xx, yy, zz) for xx in (-1, 0, 1) for yy in (-1, 0, 1)
           for zz in (-1, 0, 1)]
    gp = jnp.asarray(np.array(pts, dtype=np.float32))
    offsets = jnp.linspace(0.0, CUTOFF, NR).astype(f32)
    coeff = (-0.5 / (offsets[1] - offsets[0]) ** 2).reshape(1, 1)
    offs = offsets.reshape(NR, 1)

    const = lambda shape: pl.BlockSpec(shape, lambda g: (0,) * len(shape))
    in_specs = [
        pl.BlockSpec((1, 1, D), lambda g: (g, 0, 0)),
        pl.BlockSpec((NPG, C), lambda g: (g, 0)),
        const((NG, C)),
        const((NR, 1)),
        const((1, 1)),
        const((D, NG * D)),
        const((1, NG * D)),
        const((L, D, D)), const((L, 1, D)),
        const((L, D, D)), const((L, 1, D)),
        const((L, D, D)), const((L, 1, D)),
        const((L, NR, D)), const((L, 1, D)),
        const((L, D, D)), const((L, 1, D)),
        const((L, D, D)), const((L, 1, D)),
        const((D, NT)), const((1, NT)),
    ]
    out = pl.pallas_call(
        _decoder_body,
        grid=(G,),
        in_specs=in_specs,
        out_specs=pl.BlockSpec((NPG, C + NT), lambda g: (g, 0)),
        out_shape=jax.ShapeDtypeStruct((G * NPG, C + NT), f32),
    )(encoding.reshape(G, 1, D), pos, gp, offs, coeff, Wup_r, bup_r, Wq, bq_r, Wk, bk_r,
      Wv, bv_r, We, be_r, Wskip, bskip_r, Wfc, bfc_r, Wout, bout_r)
    return out


# speedup vs baseline: 2.5764x; 1.0274x over previous
import jax
import jax.numpy as jnp
import numpy as np
from jax.experimental import pallas as pl

G, NPG, D, C = 16, 128, 128, 3
NG, NR, H, HD, L = 27, 50, 4, 32, 2
CUTOFF = 2.0
NT = 10

_PREC = jax.lax.Precision.DEFAULT


def _dot(a, b, dims):
    return jax.lax.dot_general(
        a, b, (dims, ((), ())), precision=_PREC,
        preferred_element_type=jnp.float32)


GPP = 2


def _decoder_body(enc_ref, pos_ref, gp_ref, offs_ref, coef_ref, wup_ref,
                  bup_ref, wq_ref, bq_ref, wk_ref, bk_ref, wv_ref, bv_ref,
                  we_ref, be_ref, wskip_ref, bskip_ref, wfc_ref, bfc_ref,
                  wout_ref, bout_ref, out_ref):
    for gg in range(GPP):
        _one_graph(enc_ref[gg].reshape(1, D),
                   pos_ref[gg * NPG:(gg + 1) * NPG, :],
                   gp_ref, offs_ref, coef_ref, wup_ref, bup_ref, wq_ref,
                   bq_ref, wk_ref, bk_ref, wv_ref, bv_ref, we_ref, be_ref,
                   wskip_ref, bskip_ref, wfc_ref, bfc_ref, wout_ref,
                   bout_ref, out_ref, gg)


def _one_graph(enc, pos, gp_ref, offs_ref, coef_ref, wup_ref,
               bup_ref, wq_ref, bq_ref, wk_ref, bk_ref, wv_ref, bv_ref,
               we_ref, be_ref, wskip_ref, bskip_ref, wfc_ref, bfc_ref,
               wout_ref, bout_ref, out_ref, gg):
    f32 = jnp.float32
    gelu = jax.nn.gelu

    gp = gp_ref[...]

    gridf = gelu(_dot(enc, wup_ref[...], ((1,), (0,)))
                 + bup_ref[...]).reshape(NG, D)

    gp_t = gp.T
    d2 = ((pos[:, 0:1] - gp_t[0:1, :]) ** 2
          + (pos[:, 1:2] - gp_t[1:2, :]) ** 2
          + (pos[:, 2:3] - gp_t[2:3, :]) ** 2)
    cols = jax.lax.broadcasted_iota(jnp.int32, (NPG, NG), 1).astype(f32)
    d2m = d2
    sels, ws = [], []
    for _ in range(3):
        m = jnp.min(d2m, axis=1, keepdims=True)
        cand = jnp.where(d2m == m, cols, float(NG))
        fidx = jnp.min(cand, axis=1, keepdims=True)
        sel = cols == fidx
        sels.append(sel)
        ws.append(1.0 / (m + 1e-16))
        d2m = jnp.where(sel, 1e30, d2m)
    wtot = ws[0] + ws[1] + ws[2]
    woh = (sels[0].astype(f32) * (ws[0] / wtot)
           + sels[1].astype(f32) * (ws[1] / wtot)
           + sels[2].astype(f32) * (ws[2] / wtot))
    x = gelu(_dot(woh, gridf, ((1,), (0,))))

    offs3 = offs_ref[...].reshape(NR, 1, 1)
    coef3 = coef_ref[...].reshape(1, 1, 1)
    ii = jax.lax.broadcasted_iota(jnp.int32, (NPG, NPG), 0)
    jj = jax.lax.broadcasted_iota(jnp.int32, (NPG, NPG), 1)
    offdiag = ii != jj
    inv_sqrt_hd = 1.0 / np.sqrt(1.0 * HD)

    for l in range(L):
        pos_t = pos.T
        dT2 = ((pos[:, 0:1] - pos_t[0:1, :]) ** 2
               + (pos[:, 1:2] - pos_t[1:2, :]) ** 2
               + (pos[:, 2:3] - pos_t[2:3, :]) ** 2)
        distT = jnp.sqrt(dT2 + 1e-12)
        maskT = (distT < CUTOFF) & offdiag
        maskTf = maskT.astype(f32)
        rbf = jnp.exp(coef3 * (distT[None, :, :] - offs3) ** 2)

        xg = x
        q = _dot(xg, wq_ref[l], ((1,), (0,))) + bq_ref[l]
        k_ = _dot(xg, wk_ref[l], ((1,), (0,))) + bk_ref[l]
        v_ = _dot(xg, wv_ref[l], ((1,), (0,))) + bv_ref[l]
        skip = _dot(xg, wskip_ref[l], ((1,), (0,))) + bskip_ref[l]
        we_l = we_ref[l]
        be_l = be_ref[l]

        heads = []
        for h in range(H):
            sl = slice(h * HD, (h + 1) * HD)
            qh, kh, vh = q[:, sl], k_[:, sl], v_[:, sl]
            weh = we_l[:, sl]
            beh = be_l[:, sl]

            l1T = _dot(kh, qh, ((1,), (1,)))
            aqT = _dot(weh, qh, ((1,), (1,)))
            l2T = jnp.sum(rbf * aqT[:, None, :], axis=0)
            qbeT = _dot(beh, qh, ((1,), (1,)))
            lg = (l1T + l2T + qbeT) * inv_sqrt_hd
            lg = jnp.where(maskT, lg, -1e9)
            lmax = jnp.max(lg, axis=0, keepdims=True)
            ex = jnp.exp(lg - lmax) * maskTf
            den = jnp.sum(ex, axis=0, keepdims=True) + 1e-16
            alT = ex / den

            o1 = _dot(alT, vh, ((0,), (0,)))
            sh = jnp.sum(rbf * alT[None, :, :], axis=1)
            o2 = _dot(sh, weh, ((0,), (0,)))
            asum = _dot(alT, jnp.ones((NPG, 1), f32), ((0,), (0,)))
            heads.append(o1 + o2 + asum * beh)
        attn = jnp.concatenate(heads, axis=1)

        out = attn + skip
        hh = gelu(_dot(out, wfc_ref[l], ((1,), (0,))) + bfc_ref[l])
        xg = xg + hh
        pos = pos + xg[:, :C]
        x = jnp.concatenate([xg[:, :D - C], pos], axis=1)

    proj = _dot(x, wout_ref[...], ((1,), (0,))) + bout_ref[...]
    out_ref[gg * NPG:(gg + 1) * NPG, :] = jnp.concatenate([pos, proj], axis=1)


@jax.jit
def kernel(encoding, pos, batch, graph_sizes, Wup, bup, Wq, bq, Wk, bk,
           Wv, bv, We, be, Wskip, bskip, Wfc, bfc, Wout, bout):
    del batch, graph_sizes
    f32 = jnp.float32

    Wup_r = Wup.reshape(D, D, NG).transpose(0, 2, 1).reshape(D, NG * D)
    bup_r = bup.reshape(D, NG).T.reshape(1, NG * D)
    bq_r = bq.reshape(L, 1, D)
    bk_r = bk.reshape(L, 1, D)
    bv_r = bv.reshape(L, 1, D)
    be_r = be.reshape(L, 1, D)
    bskip_r = bskip.reshape(L, 1, D)
    bfc_r = bfc.reshape(L, 1, D)
    bout_r = bout.reshape(1, NT)

    pts = [(xx, yy, zz) for xx in (-1, 0, 1) for yy in (-1, 0, 1)
           for zz in (-1, 0, 1)]
    gp = jnp.asarray(np.array(pts, dtype=np.float32))
    offsets = jnp.linspace(0.0, CUTOFF, NR).astype(f32)
    coeff = (-0.5 / (offsets[1] - offsets[0]) ** 2).reshape(1, 1)
    offs = offsets.reshape(NR, 1)

    const = lambda shape: pl.BlockSpec(shape, lambda g: (0,) * len(shape))
    in_specs = [
        pl.BlockSpec((GPP, 1, D), lambda g: (g, 0, 0)),
        pl.BlockSpec((GPP * NPG, C), lambda g: (g, 0)),
        const((NG, C)),
        const((NR, 1)),
        const((1, 1)),
        const((D, NG * D)),
        const((1, NG * D)),
        const((L, D, D)), const((L, 1, D)),
        const((L, D, D)), const((L, 1, D)),
        const((L, D, D)), const((L, 1, D)),
        const((L, NR, D)), const((L, 1, D)),
        const((L, D, D)), const((L, 1, D)),
        const((L, D, D)), const((L, 1, D)),
        const((D, NT)), const((1, NT)),
    ]
    out = pl.pallas_call(
        _decoder_body,
        grid=(G // GPP,),
        in_specs=in_specs,
        out_specs=pl.BlockSpec((GPP * NPG, C + NT), lambda g: (g, 0)),
        out_shape=jax.ShapeDtypeStruct((G * NPG, C + NT), f32),
    )(encoding.reshape(G, 1, D), pos, gp, offs, coeff, Wup_r, bup_r, Wq, bq_r, Wk, bk_r,
      Wv, bv_r, We, be_r, Wskip, bskip_r, Wfc, bfc_r, Wout, bout_r)
    return out


# speedup vs baseline: 2.8945x; 1.1235x over previous
import jax
import jax.numpy as jnp
import numpy as np
from jax.experimental import pallas as pl

G, NPG, D, C = 16, 128, 128, 3
NG, NR, H, HD, L = 27, 50, 4, 32, 2
CUTOFF = 2.0
NT = 10

_PREC = jax.lax.Precision.DEFAULT


def _dot(a, b, dims):
    return jax.lax.dot_general(
        a, b, (dims, ((), ())), precision=_PREC,
        preferred_element_type=jnp.float32)


GPP = 2


def _decoder_body(enc_ref, pos_ref, gp_ref, offs_ref, coef_ref, wup_ref,
                  bup_ref, wq_ref, bq_ref, wk_ref, bk_ref, wv_ref, bv_ref,
                  we_ref, be_ref, wskip_ref, bskip_ref, wfc_ref, bfc_ref,
                  wout_ref, bout_ref, out_ref):
    for gg in range(GPP):
        _one_graph(enc_ref[gg].reshape(1, D),
                   pos_ref[gg * NPG:(gg + 1) * NPG, :],
                   gp_ref, offs_ref, coef_ref, wup_ref, bup_ref, wq_ref,
                   bq_ref, wk_ref, bk_ref, wv_ref, bv_ref, we_ref, be_ref,
                   wskip_ref, bskip_ref, wfc_ref, bfc_ref, wout_ref,
                   bout_ref, out_ref, gg)


def _one_graph(enc, pos, gp_ref, offs_ref, coef_ref, wup_ref,
               bup_ref, wq_ref, bq_ref, wk_ref, bk_ref, wv_ref, bv_ref,
               we_ref, be_ref, wskip_ref, bskip_ref, wfc_ref, bfc_ref,
               wout_ref, bout_ref, out_ref, gg):
    f32 = jnp.float32
    gelu = jax.nn.gelu

    gp = gp_ref[...]

    gridf = gelu(_dot(enc, wup_ref[...], ((1,), (0,)))
                 + bup_ref[...]).reshape(NG, D)

    gp_t = gp.T
    d2 = ((pos[:, 0:1] - gp_t[0:1, :]) ** 2
          + (pos[:, 1:2] - gp_t[1:2, :]) ** 2
          + (pos[:, 2:3] - gp_t[2:3, :]) ** 2)
    cols = jax.lax.broadcasted_iota(jnp.int32, (NPG, NG), 1).astype(f32)
    d2m = d2
    sels, ws = [], []
    for _ in range(3):
        m = jnp.min(d2m, axis=1, keepdims=True)
        cand = jnp.where(d2m == m, cols, float(NG))
        fidx = jnp.min(cand, axis=1, keepdims=True)
        sel = cols == fidx
        sels.append(sel)
        ws.append(1.0 / (m + 1e-16))
        d2m = jnp.where(sel, 1e30, d2m)
    wtot = ws[0] + ws[1] + ws[2]
    woh = (sels[0].astype(f32) * (ws[0] / wtot)
           + sels[1].astype(f32) * (ws[1] / wtot)
           + sels[2].astype(f32) * (ws[2] / wtot))
    x = gelu(_dot(woh, gridf, ((1,), (0,))))

    offs3 = offs_ref[...].reshape(NR, 1, 1)
    c2exp = coef_ref[...].reshape(1, 1, 1) * np.float32(1.4426950408889634)
    ii = jax.lax.broadcasted_iota(jnp.int32, (NPG, NPG), 0)
    jj = jax.lax.broadcasted_iota(jnp.int32, (NPG, NPG), 1)
    offdiag = ii != jj
    inv_sqrt_hd = 1.0 / np.sqrt(1.0 * HD)

    for l in range(L):
        pos_t = pos.T
        dT2 = ((pos[:, 0:1] - pos_t[0:1, :]) ** 2
               + (pos[:, 1:2] - pos_t[1:2, :]) ** 2
               + (pos[:, 2:3] - pos_t[2:3, :]) ** 2)
        distT = jnp.sqrt(dT2 + 1e-12)
        maskT = (distT < CUTOFF) & offdiag
        maskTf = maskT.astype(f32)
        rbf = jnp.exp2(c2exp * (distT[None, :, :] - offs3) ** 2)

        xg = x
        q = _dot(xg, wq_ref[l], ((1,), (0,))) + bq_ref[l]
        k_ = _dot(xg, wk_ref[l], ((1,), (0,))) + bk_ref[l]
        v_ = _dot(xg, wv_ref[l], ((1,), (0,))) + bv_ref[l]
        skip = _dot(xg, wskip_ref[l], ((1,), (0,))) + bskip_ref[l]
        we_l = we_ref[l]
        be_l = be_ref[l]

        qs = [q[:, h * HD:(h + 1) * HD] for h in range(H)]
        aqTs = [_dot(we_l[:, h * HD:(h + 1) * HD], qs[h], ((1,), (1,)))
                for h in range(H)]

        JC = 64
        l2Ts = [[] for _ in range(H)]
        for jb in range(0, NPG, JC):
            accs = [rbf[0, jb:jb + JC, :] * aqTs[h][0:1, :] for h in range(H)]
            for r in range(1, NR):
                slab = rbf[r, jb:jb + JC, :]
                for h in range(H):
                    accs[h] = accs[h] + slab * aqTs[h][r:r + 1, :]
            for h in range(H):
                l2Ts[h].append(accs[h])

        alTs = []
        for h in range(H):
            sl = slice(h * HD, (h + 1) * HD)
            qh, kh = qs[h], k_[:, sl]
            l1T = _dot(kh, qh, ((1,), (1,)))
            l2T = jnp.concatenate(l2Ts[h], axis=0)
            qbeT = _dot(be_l[:, sl], qh, ((1,), (1,)))
            lg = (l1T + l2T + qbeT) * inv_sqrt_hd
            lg = jnp.where(maskT, lg, -1e9)
            lmax = jnp.max(lg, axis=0, keepdims=True)
            ex = jnp.exp(lg - lmax) * maskTf
            den = jnp.sum(ex, axis=0, keepdims=True) + 1e-16
            alTs.append(ex / den)

        srows = [[None] * NR for _ in range(H)]
        for jb in range(0, NPG, JC):
            for r in range(NR):
                slab = rbf[r, jb:jb + JC, :]
                for h in range(H):
                    row = jnp.sum(slab * alTs[h][jb:jb + JC, :], axis=0,
                                  keepdims=True)
                    srows[h][r] = (row if srows[h][r] is None
                                   else srows[h][r] + row)

        heads = []
        for h in range(H):
            sl = slice(h * HD, (h + 1) * HD)
            vh = v_[:, sl]
            weh = we_l[:, sl]
            beh = be_l[:, sl]
            alT = alTs[h]
            o1 = _dot(alT, vh, ((0,), (0,)))
            sh = jnp.concatenate(srows[h], axis=0)
            o2 = _dot(sh, weh, ((0,), (0,)))
            asum = _dot(alT, jnp.ones((NPG, 1), f32), ((0,), (0,)))
            heads.append(o1 + o2 + asum * beh)
        attn = jnp.concatenate(heads, axis=1)

        out = attn + skip
        hh = gelu(_dot(out, wfc_ref[l], ((1,), (0,))) + bfc_ref[l])
        xg = xg + hh
        pos = pos + xg[:, :C]
        x = jnp.concatenate([xg[:, :D - C], pos], axis=1)

    proj = _dot(x, wout_ref[...], ((1,), (0,))) + bout_ref[...]
    out_ref[gg * NPG:(gg + 1) * NPG, :] = jnp.concatenate([pos, proj], axis=1)


@jax.jit
def kernel(encoding, pos, batch, graph_sizes, Wup, bup, Wq, bq, Wk, bk,
           Wv, bv, We, be, Wskip, bskip, Wfc, bfc, Wout, bout):
    del batch, graph_sizes
    f32 = jnp.float32

    Wup_r = Wup.reshape(D, D, NG).transpose(0, 2, 1).reshape(D, NG * D)
    bup_r = bup.reshape(D, NG).T.reshape(1, NG * D)
    bq_r = bq.reshape(L, 1, D)
    bk_r = bk.reshape(L, 1, D)
    bv_r = bv.reshape(L, 1, D)
    be_r = be.reshape(L, 1, D)
    bskip_r = bskip.reshape(L, 1, D)
    bfc_r = bfc.reshape(L, 1, D)
    bout_r = bout.reshape(1, NT)

    pts = [(xx, yy, zz) for xx in (-1, 0, 1) for yy in (-1, 0, 1)
           for zz in (-1, 0, 1)]
    gp = jnp.asarray(np.array(pts, dtype=np.float32))
    offsets = jnp.linspace(0.0, CUTOFF, NR).astype(f32)
    coeff = (-0.5 / (offsets[1] - offsets[0]) ** 2).reshape(1, 1)
    offs = offsets.reshape(NR, 1)

    const = lambda shape: pl.BlockSpec(shape, lambda g: (0,) * len(shape))
    in_specs = [
        pl.BlockSpec((GPP, 1, D), lambda g: (g, 0, 0)),
        pl.BlockSpec((GPP * NPG, C), lambda g: (g, 0)),
        const((NG, C)),
        const((NR, 1)),
        const((1, 1)),
        const((D, NG * D)),
        const((1, NG * D)),
        const((L, D, D)), const((L, 1, D)),
        const((L, D, D)), const((L, 1, D)),
        const((L, D, D)), const((L, 1, D)),
        const((L, NR, D)), const((L, 1, D)),
        const((L, D, D)), const((L, 1, D)),
        const((L, D, D)), const((L, 1, D)),
        const((D, NT)), const((1, NT)),
    ]
    out = pl.pallas_call(
        _decoder_body,
        grid=(G // GPP,),
        in_specs=in_specs,
        out_specs=pl.BlockSpec((GPP * NPG, C + NT), lambda g: (g, 0)),
        out_shape=jax.ShapeDtypeStruct((G * NPG, C + NT), f32),
    )(encoding.reshape(G, 1, D), pos, gp, offs, coeff, Wup_r, bup_r, Wq, bq_r, Wk, bk_r,
      Wv, bv_r, We, be_r, Wskip, bskip_r, Wfc, bfc_r, Wout, bout_r)
    return out
